# baseline (speedup 1.0000x reference)
import functools

import jax
import jax.numpy as jnp
from jax import lax
from jax.experimental import pallas as pl
from jax.experimental.pallas import tpu as pltpu
from jax.experimental.pallas import tpu_sc as plsc

N = 10000
E = 320000
D = 128

NC = 2
NS = 16
NW = NC * NS
B = 128
CH = 80
HC = 40
EW = CH * B
E_PAD = NW * EW
ACC_ROWS = NS * 640


def _agg_body(with_cnt, *refs):
    if with_cnt:
        (x_hbm, src_hbm, dst_hbm, out_hbm, cnt_hbm,
         src_v, dst_v, rows_v, rows_w, ones_v, zc_v, acc_sh, cnt_sh,
         sem_a, sem_b) = refs
    else:
        (x_hbm, src_hbm, dst_hbm, out_hbm,
         src_v, dst_v, rows_v, rows_w, ones_v, zc_v, acc_sh, cnt_sh,
         sem_a, sem_b) = refs
        cnt_hbm = None
    cid = lax.axis_index("c")
    sid = lax.axis_index("s")
    wid = sid * NC + cid

    z16 = jnp.zeros((16,), jnp.float32)

    def _zero_rows(r, _):
        for c in range(D // 16):
            rows_v[r, pl.ds(c * 16, 16)] = z16
        return 0
    lax.fori_loop(0, B, _zero_rows, 0)

    def _zero_zc(r, _):
        zc_v[pl.ds(r * 16, 16)] = z16
        return 0
    lax.fori_loop(0, 640 // 16, _zero_zc, 0)
    for c in range(B // 16):
        ones_v[pl.ds(c * 16, 16)] = jnp.ones((16,), jnp.float32)

    for k in range(5):
        pltpu.sync_copy(rows_v, acc_sh.at[pl.ds(sid * 640 + k * B, B)])
    pltpu.sync_copy(zc_v, cnt_sh.at[pl.ds(sid * 640, 640)])

    zi16 = jnp.zeros((16,), jnp.int32)
    for c in range(B // 16):
        src_v[HC, pl.ds(c * 16, 16)] = zi16

    plsc.subcore_barrier()

    for h in range(CH // HC):
        pltpu.sync_copy(src_hbm.at[wid, pl.ds(h * HC, HC)],
                        src_v.at[pl.ds(0, HC)])
        pltpu.sync_copy(dst_hbm.at[wid, pl.ds(h * HC, HC)], dst_v)
        pltpu.async_copy(x_hbm.at[src_v.at[0]], rows_v, sem_a)

        @pl.loop(0, HC, step=2)
        def _chunks(j):
            pltpu.async_copy(x_hbm.at[src_v.at[j + 1]], rows_w, sem_b)
            pltpu.make_async_copy(x_hbm.at[src_v.at[j]], rows_v, sem_a).wait()
            pltpu.sync_copy(rows_v, acc_sh.at[dst_v.at[j]], add=True)
            if with_cnt:
                pltpu.sync_copy(ones_v, cnt_sh.at[dst_v.at[j]], add=True)
            pltpu.async_copy(x_hbm.at[src_v.at[j + 2]], rows_v, sem_a)
            pltpu.make_async_copy(x_hbm.at[src_v.at[j + 1]], rows_w, sem_b).wait()
            pltpu.sync_copy(rows_w, acc_sh.at[dst_v.at[j + 1]], add=True)
            if with_cnt:
                pltpu.sync_copy(ones_v, cnt_sh.at[dst_v.at[j + 1]], add=True)

        pltpu.make_async_copy(x_hbm.at[src_v.at[HC]], rows_v, sem_a).wait()

    plsc.subcore_barrier()

    pltpu.sync_copy(acc_sh.at[pl.ds(sid * 640, 640)],
                    out_hbm.at[cid, pl.ds(sid * 640, 640)])
    if with_cnt:
        pltpu.sync_copy(cnt_sh.at[pl.ds(sid * 640, 640)],
                        cnt_hbm.at[cid, pl.ds(sid * 640, 640)])


def _make_agg(with_cnt):
    out_type = [jax.ShapeDtypeStruct((NC, ACC_ROWS, D), jnp.float32)]
    if with_cnt:
        out_type.append(jax.ShapeDtypeStruct((NC, ACC_ROWS), jnp.float32))
    return pl.kernel(
        functools.partial(_agg_body, with_cnt),
        out_type=tuple(out_type) if with_cnt else out_type[0],
        mesh=plsc.VectorSubcoreMesh(core_axis_name="c", subcore_axis_name="s"),
        scratch_types=[
            pltpu.VMEM((HC + 1, B), jnp.int32),
            pltpu.VMEM((HC, B), jnp.int32),
            pltpu.VMEM((B, D), jnp.float32),
            pltpu.VMEM((B, D), jnp.float32),
            pltpu.VMEM((B,), jnp.float32),
            pltpu.VMEM((640,), jnp.float32),
            pltpu.VMEM_SHARED((ACC_ROWS, D), jnp.float32),
            pltpu.VMEM_SHARED((ACC_ROWS,), jnp.float32),
            pltpu.SemaphoreType.DMA,
            pltpu.SemaphoreType.DMA,
        ],
    )


_agg_with_cnt = _make_agg(True)
_agg_no_cnt = _make_agg(False)


def _combine_body(relu, agg_ref, cnt_ref, x_ref, wl_ref, b_ref, wr_ref, o_ref):
    acc = agg_ref[0] + agg_ref[1]
    c = cnt_ref[0] + cnt_ref[1]
    inv = 1.0 / jnp.maximum(c, 1.0)
    mean = acc * inv[:, None]
    h = lax.dot_general(mean, wl_ref[...], (((1,), (1,)), ((), ())),
                        preferred_element_type=jnp.float32)
    h = h + b_ref[...]
    h = h + lax.dot_general(x_ref[...], wr_ref[...], (((1,), (1,)), ((), ())),
                            preferred_element_type=jnp.float32)
    o_ref[...] = jnp.maximum(h, 0.0) if relu else h


_RB = 2048


def _combine(agg, cnt, x, W_l, b_l, W_r, relu):
    grid = (ACC_ROWS // _RB,)
    return pl.pallas_call(
        functools.partial(_combine_body, relu),
        grid=grid,
        in_specs=[
            pl.BlockSpec((NC, _RB, D), lambda i: (0, i, 0)),
            pl.BlockSpec((NC, _RB), lambda i: (0, i)),
            pl.BlockSpec((_RB, D), lambda i: (i, 0)),
            pl.BlockSpec((D, D), lambda i: (0, 0)),
            pl.BlockSpec((1, D), lambda i: (0, 0)),
            pl.BlockSpec((D, D), lambda i: (0, 0)),
        ],
        out_specs=pl.BlockSpec((_RB, D), lambda i: (i, 0)),
        out_shape=jax.ShapeDtypeStruct((N, D), jnp.float32),
    )(agg, cnt, x, W_l, b_l.reshape(1, D), W_r)


def kernel(x, edge_index, W_l0, b_l0, W_r0, W_l1, b_l1, W_r1):
    src = edge_index[0]
    dst = edge_index[1]
    pad = E_PAD - E
    src_p = jnp.concatenate(
        [src, jnp.zeros((pad,), jnp.int32)]).reshape(NW, CH, B)
    dst_p = jnp.concatenate(
        [dst, jnp.full((pad,), N, jnp.int32)]).reshape(NW, CH, B)

    agg0, cnt = _agg_with_cnt(x, src_p, dst_p)
    a1 = _combine(agg0, cnt, x, W_l0, b_l0, W_r0, relu=True)
    agg1 = _agg_no_cnt(a1, src_p, dst_p)
    return _combine(agg1, cnt, a1, W_l1, b_l1, W_r1, relu=False)


# speedup vs baseline: 4.4827x; 4.4827x over previous
import functools

import jax
import jax.numpy as jnp
from jax import lax
from jax.experimental import pallas as pl
from jax.experimental.pallas import tpu as pltpu
from jax.experimental.pallas import tpu_sc as plsc

N = 10000
E = 320000
D = 128

NC = 2
NS = 16
B = 128
CH0 = 80
CH1 = 80
HC = 40
NCHUNK = NS * (CH0 + CH1)
E_PAD = NCHUNK * B
ACC_ROWS = NS * 640


def _agg_body(with_cnt, *refs):
    if with_cnt:
        (x_hbm, src_hbm, dst_hbm, out_hbm, cnt_hbm,
         src_v, dst_v, rows_v, rows_w, ones_v, zc_v, acc_sh, cnt_sh,
         sem_a, sem_b, sem_c, sem_d, sem_e) = refs
    else:
        (x_hbm, src_hbm, dst_hbm, out_hbm,
         src_v, dst_v, rows_v, rows_w, ones_v, zc_v, acc_sh, cnt_sh,
         sem_a, sem_b, sem_c, sem_d, sem_e) = refs
        cnt_hbm = None
    cid = lax.axis_index("c")
    sid = lax.axis_index("s")

    z16 = jnp.zeros((16,), jnp.float32)

    def _zero_rows(r, _):
        for c in range(D // 16):
            rows_v[r, pl.ds(c * 16, 16)] = z16
        return 0
    lax.fori_loop(0, B, _zero_rows, 0)

    def _zero_zc(r, _):
        zc_v[pl.ds(r * 16, 16)] = z16
        return 0
    lax.fori_loop(0, 640 // 16, _zero_zc, 0)
    for c in range(B // 16):
        ones_v[pl.ds(c * 16, 16)] = jnp.ones((16,), jnp.float32)

    for k in range(5):
        pltpu.sync_copy(rows_v, acc_sh.at[pl.ds(sid * 640 + k * B, B)])
    pltpu.sync_copy(zc_v, cnt_sh.at[pl.ds(sid * 640, 640)])

    plsc.subcore_barrier()

    def _run_chunks(nch, base):
        for h in range(nch // HC):
            pltpu.sync_copy(src_hbm.at[pl.ds(base + h * HC, HC)], src_v)
            pltpu.sync_copy(dst_hbm.at[pl.ds(base + h * HC, HC)], dst_v)

            def _pair(i, _):
                j = 2 * i
                d0 = pltpu.async_copy(x_hbm.at[src_v.at[j]], rows_v, sem_a)
                d1 = pltpu.async_copy(x_hbm.at[src_v.at[j + 1]], rows_w, sem_b)
                d0.wait()
                s0 = pltpu.async_copy(rows_v, acc_sh.at[dst_v.at[j]], sem_c,
                                      add=True)
                d1.wait()
                s1 = pltpu.async_copy(rows_w, acc_sh.at[dst_v.at[j + 1]],
                                      sem_d, add=True)
                if with_cnt:
                    c0 = pltpu.async_copy(ones_v, cnt_sh.at[dst_v.at[j]],
                                          sem_e, add=True)
                    c1 = pltpu.async_copy(ones_v, cnt_sh.at[dst_v.at[j + 1]],
                                          sem_e, add=True)
                s0.wait()
                s1.wait()
                if with_cnt:
                    c0.wait()
                    c1.wait()
                return 0
            lax.fori_loop(0, HC // 2, _pair, 0)

    @pl.when(cid == 0)
    def _():
        _run_chunks(CH0, sid * CH0)

    @pl.when(cid == 1)
    def _():
        _run_chunks(CH1, NS * CH0 + sid * CH1)

    plsc.subcore_barrier()

    pltpu.sync_copy(acc_sh.at[pl.ds(sid * 640, 640)],
                    out_hbm.at[cid, pl.ds(sid * 640, 640)])
    if with_cnt:
        pltpu.sync_copy(cnt_sh.at[pl.ds(sid * 640, 640)],
                        cnt_hbm.at[cid, pl.ds(sid * 640, 640)])


def _make_agg(with_cnt):
    out_type = [jax.ShapeDtypeStruct((NC, ACC_ROWS, D), jnp.float32)]
    if with_cnt:
        out_type.append(jax.ShapeDtypeStruct((NC, ACC_ROWS), jnp.float32))
    return pl.kernel(
        functools.partial(_agg_body, with_cnt),
        out_type=tuple(out_type) if with_cnt else out_type[0],
        mesh=plsc.VectorSubcoreMesh(core_axis_name="c", subcore_axis_name="s"),
        scratch_types=[
            pltpu.VMEM((HC, B), jnp.int32),
            pltpu.VMEM((HC, B), jnp.int32),
            pltpu.VMEM((B, D), jnp.float32),
            pltpu.VMEM((B, D), jnp.float32),
            pltpu.VMEM((B,), jnp.float32),
            pltpu.VMEM((640,), jnp.float32),
            pltpu.VMEM_SHARED((ACC_ROWS, D), jnp.float32),
            pltpu.VMEM_SHARED((ACC_ROWS,), jnp.float32),
            pltpu.SemaphoreType.DMA,
            pltpu.SemaphoreType.DMA,
            pltpu.SemaphoreType.DMA,
            pltpu.SemaphoreType.DMA,
            pltpu.SemaphoreType.DMA,
        ],
    )


_agg_with_cnt = _make_agg(True)
_agg_no_cnt = _make_agg(False)


def _combine_body(relu, agg_ref, cnt_ref, x_ref, wl_ref, b_ref, wr_ref, o_ref):
    acc = agg_ref[0] + agg_ref[1]
    c = cnt_ref[0] + cnt_ref[1]
    inv = 1.0 / jnp.maximum(c, 1.0)
    mean = acc * inv[:, None]
    h = lax.dot_general(mean, wl_ref[...], (((1,), (1,)), ((), ())),
                        preferred_element_type=jnp.float32)
    h = h + b_ref[...]
    h = h + lax.dot_general(x_ref[...], wr_ref[...], (((1,), (1,)), ((), ())),
                            preferred_element_type=jnp.float32)
    o_ref[...] = jnp.maximum(h, 0.0) if relu else h


_RB = 2048


def _combine(agg, cnt, x, W_l, b_l, W_r, relu):
    grid = (ACC_ROWS // _RB,)
    return pl.pallas_call(
        functools.partial(_combine_body, relu),
        grid=grid,
        in_specs=[
            pl.BlockSpec((NC, _RB, D), lambda i: (0, i, 0)),
            pl.BlockSpec((NC, _RB), lambda i: (0, i)),
            pl.BlockSpec((_RB, D), lambda i: (i, 0)),
            pl.BlockSpec((D, D), lambda i: (0, 0)),
            pl.BlockSpec((1, D), lambda i: (0, 0)),
            pl.BlockSpec((D, D), lambda i: (0, 0)),
        ],
        out_specs=pl.BlockSpec((_RB, D), lambda i: (i, 0)),
        out_shape=jax.ShapeDtypeStruct((N, D), jnp.float32),
    )(agg, cnt, x, W_l, b_l.reshape(1, D), W_r)


def kernel(x, edge_index, W_l0, b_l0, W_r0, W_l1, b_l1, W_r1):
    src = edge_index[0]
    dst = edge_index[1]
    pad = E_PAD - E
    pad_src = jnp.arange(pad, dtype=jnp.int32) % N
    pad_dst = N + jnp.arange(pad, dtype=jnp.int32) % (ACC_ROWS - N)
    src_p = jnp.concatenate([src, pad_src]).reshape(NCHUNK, B)
    dst_p = jnp.concatenate([dst, pad_dst]).reshape(NCHUNK, B)

    agg0, cnt = _agg_with_cnt(x, src_p, dst_p)
    a1 = _combine(agg0, cnt, x, W_l0, b_l0, W_r0, relu=True)
    agg1 = _agg_no_cnt(a1, src_p, dst_p)
    return _combine(agg1, cnt, a1, W_l1, b_l1, W_r1, relu=False)


# speedup vs baseline: 5.0113x; 1.1179x over previous
import functools

import jax
import jax.numpy as jnp
from jax import lax
from jax.experimental import pallas as pl
from jax.experimental.pallas import tpu as pltpu
from jax.experimental.pallas import tpu_sc as plsc

N = 10000
E = 320000
D = 128

NC = 2
NS = 16
B = 64
CH0 = 160
CH1 = 160
HC = 16
NB = 4
NCHUNK = NS * (CH0 + CH1)
E_PAD = NCHUNK * B
ACC_ROWS = NS * 640


def _agg_body(with_cnt, *refs):
    if with_cnt:
        (x_hbm, src_hbm, dst_hbm, out_hbm, cnt_hbm,
         sv0, sv1, dv0, dv1, r0, r1, r2, r3, ones_v, zc_v, acc_sh,
         cnt_sh, g0, g1, g2, g3, s0, s1, s2, s3, csem,
         i0, i1, i2, i3) = refs
    else:
        (x_hbm, src_hbm, dst_hbm, out_hbm,
         sv0, sv1, dv0, dv1, r0, r1, r2, r3, ones_v, zc_v, acc_sh,
         cnt_sh, g0, g1, g2, g3, s0, s1, s2, s3, csem,
         i0, i1, i2, i3) = refs
        cnt_hbm = None
    sv = (sv0, sv1)
    dv = (dv0, dv1)
    isem_s = (i0, i1)
    isem_d = (i2, i3)
    rows = (r0, r1, r2, r3)
    gsem = (g0, g1, g2, g3)
    ssem = (s0, s1, s2, s3)
    rows_v = r0
    cid = lax.axis_index("c")
    sid = lax.axis_index("s")

    z16 = jnp.zeros((16,), jnp.float32)

    def _zero_rows(r, _):
        for c in range(D // 16):
            rows_v[r, pl.ds(c * 16, 16)] = z16
        return 0
    lax.fori_loop(0, B, _zero_rows, 0)

    def _zero_zc(r, _):
        zc_v[pl.ds(r * 16, 16)] = z16
        return 0
    lax.fori_loop(0, 640 // 16, _zero_zc, 0)
    for c in range(B // 16):
        ones_v[pl.ds(c * 16, 16)] = jnp.ones((16,), jnp.float32)

    for k in range(640 // B):
        pltpu.sync_copy(rows_v, acc_sh.at[pl.ds(sid * 640 + k * B, B)])
    pltpu.sync_copy(zc_v, cnt_sh.at[pl.ds(sid * 640, 640)])

    plsc.subcore_barrier()

    def _gather(src_v, j, b):
        return pltpu.async_copy(x_hbm.at[src_v.at[j]], rows[b], gsem[b])

    def _scatter(dst_v, j, b, cds):
        s = pltpu.async_copy(rows[b], acc_sh.at[dst_v.at[j]], ssem[b],
                             add=True)
        if with_cnt:
            cds.append(pltpu.async_copy(ones_v, cnt_sh.at[dst_v.at[j]],
                                        csem, add=True))
        return s

    def _run_chunks(nch, base):
        nblk = nch // HC

        def _stage(h, p):
            return (
                pltpu.async_copy(src_hbm.at[pl.ds(base + h * HC, HC)],
                                 sv[p], isem_s[p]),
                pltpu.async_copy(dst_hbm.at[pl.ds(base + h * HC, HC)],
                                 dv[p], isem_d[p]),
            )

        pend = _stage(0, 0)
        for h in range(nblk):
            p = h % 2
            pend[0].wait()
            pend[1].wait()
            if h + 1 < nblk:
                pend = _stage(h + 1, 1 - p)
            src_v = sv[p]
            dst_v = dv[p]

            def _ring(i, _):
                j = 2 * NB * i
                cds = []
                gds = [_gather(src_v, j + b, b) for b in range(NB)]
                sds = []
                for b in range(NB):
                    gds[b].wait()
                    sds.append(_scatter(dst_v, j + b, b, cds))
                for b in range(NB):
                    sds[b].wait()
                    gds[b] = _gather(src_v, j + NB + b, b)
                sds = []
                for b in range(NB):
                    gds[b].wait()
                    sds.append(_scatter(dst_v, j + NB + b, b, cds))
                for b in range(NB):
                    sds[b].wait()
                for c in cds:
                    c.wait()
                return 0
            lax.fori_loop(0, HC // (2 * NB), _ring, 0)

    @pl.when(cid == 0)
    def _():
        _run_chunks(CH0, sid * CH0)

    @pl.when(cid == 1)
    def _():
        _run_chunks(CH1, NS * CH0 + sid * CH1)

    plsc.subcore_barrier()

    pltpu.sync_copy(acc_sh.at[pl.ds(sid * 640, 640)],
                    out_hbm.at[cid, pl.ds(sid * 640, 640)])
    if with_cnt:
        pltpu.sync_copy(cnt_sh.at[pl.ds(sid * 640, 640)],
                        cnt_hbm.at[cid, pl.ds(sid * 640, 640)])


def _make_agg(with_cnt):
    out_type = [jax.ShapeDtypeStruct((NC, ACC_ROWS, D), jnp.float32)]
    if with_cnt:
        out_type.append(jax.ShapeDtypeStruct((NC, ACC_ROWS), jnp.float32))
    return pl.kernel(
        functools.partial(_agg_body, with_cnt),
        out_type=tuple(out_type) if with_cnt else out_type[0],
        mesh=plsc.VectorSubcoreMesh(core_axis_name="c", subcore_axis_name="s"),
        scratch_types=[
            pltpu.VMEM((HC, B), jnp.int32),
            pltpu.VMEM((HC, B), jnp.int32),
            pltpu.VMEM((HC, B), jnp.int32),
            pltpu.VMEM((HC, B), jnp.int32),
            pltpu.VMEM((B, D), jnp.float32),
            pltpu.VMEM((B, D), jnp.float32),
            pltpu.VMEM((B, D), jnp.float32),
            pltpu.VMEM((B, D), jnp.float32),
            pltpu.VMEM((B,), jnp.float32),
            pltpu.VMEM((640,), jnp.float32),
            pltpu.VMEM_SHARED((ACC_ROWS, D), jnp.float32),
            pltpu.VMEM_SHARED((ACC_ROWS,), jnp.float32),
        ] + [pltpu.SemaphoreType.DMA] * 13,
    )


_agg_with_cnt = _make_agg(True)
_agg_no_cnt = _make_agg(False)


def _combine_body(relu, agg_ref, cnt_ref, x_ref, wl_ref, b_ref, wr_ref, o_ref):
    acc = agg_ref[0] + agg_ref[1]
    c = cnt_ref[0] + cnt_ref[1]
    inv = 1.0 / jnp.maximum(c, 1.0)
    mean = acc * inv[:, None]
    h = lax.dot_general(mean, wl_ref[...], (((1,), (1,)), ((), ())),
                        preferred_element_type=jnp.float32)
    h = h + b_ref[...]
    h = h + lax.dot_general(x_ref[...], wr_ref[...], (((1,), (1,)), ((), ())),
                            preferred_element_type=jnp.float32)
    o_ref[...] = jnp.maximum(h, 0.0) if relu else h


_RB = 2048


def _combine(agg, cnt, x, W_l, b_l, W_r, relu):
    grid = (ACC_ROWS // _RB,)
    return pl.pallas_call(
        functools.partial(_combine_body, relu),
        grid=grid,
        in_specs=[
            pl.BlockSpec((NC, _RB, D), lambda i: (0, i, 0)),
            pl.BlockSpec((NC, _RB), lambda i: (0, i)),
            pl.BlockSpec((_RB, D), lambda i: (i, 0)),
            pl.BlockSpec((D, D), lambda i: (0, 0)),
            pl.BlockSpec((1, D), lambda i: (0, 0)),
            pl.BlockSpec((D, D), lambda i: (0, 0)),
        ],
        out_specs=pl.BlockSpec((_RB, D), lambda i: (i, 0)),
        out_shape=jax.ShapeDtypeStruct((N, D), jnp.float32),
    )(agg, cnt, x, W_l, b_l.reshape(1, D), W_r)


def kernel(x, edge_index, W_l0, b_l0, W_r0, W_l1, b_l1, W_r1):
    src = edge_index[0]
    dst = edge_index[1]
    pad = E_PAD - E
    pad_src = jnp.arange(pad, dtype=jnp.int32) % N
    pad_dst = N + jnp.arange(pad, dtype=jnp.int32) % (ACC_ROWS - N)
    src_p = jnp.concatenate([src, pad_src]).reshape(NCHUNK, B)
    dst_p = jnp.concatenate([dst, pad_dst]).reshape(NCHUNK, B)

    agg0, cnt = _agg_with_cnt(x, src_p, dst_p)
    a1 = _combine(agg0, cnt, x, W_l0, b_l0, W_r0, relu=True)
    agg1 = _agg_no_cnt(a1, src_p, dst_p)
    return _combine(agg1, cnt, a1, W_l1, b_l1, W_r1, relu=False)
